# Initial kernel scaffold; baseline (speedup 1.0000x reference)
import jax
import jax.numpy as jnp
from jax.experimental import pallas as pl


def kernel(x, edge_index, edge_weight, W_in, b_in, W_h):
    raise NotImplementedError("write your pallas kernel here")


# speedup vs baseline: 9.6034x; 9.6034x over previous
import dataclasses
import functools

import jax
import jax.numpy as jnp
from jax import lax
from jax.experimental import pallas as pl
from jax.experimental.pallas import tpu as pltpu
from jax.experimental.pallas import tpu_sc as plsc

_LEAK = 0.9

_NC = 2
_NS = 16
_NW = _NC * _NS
_L = 16

def _sc_params():
    cp = pltpu.CompilerParams()
    if "use_tc_tiling_on_sc" in pltpu.CompilerParams.__dataclass_fields__:
        cp = dataclasses.replace(cp, use_tc_tiling_on_sc=False)
    return cp


_M = 80
_K = 8
_G = _M * _K


def _lane_bcast(vec, lane):
    idx = jnp.full((_L,), lane, dtype=jnp.int32)
    dnums = lax.GatherDimensionNumbers(
        offset_dims=(), collapsed_slice_dims=(0,), start_index_map=(0,))
    return lax.gather(vec, idx[:, None], dnums, slice_sizes=(1,),
                      mode=lax.GatherScatterMode.PROMISE_IN_BOUNDS)


def _propagate(col, row, ew, h):
    n, hdim = h.shape
    e = ew.shape[0]
    nblocks = e // _G
    stripe = n // _NS
    mesh = plsc.VectorSubcoreMesh(core_axis_name="c", subcore_axis_name="s")

    @functools.partial(
        pl.kernel,
        out_type=jax.ShapeDtypeStruct((_NC, n, hdim), jnp.float32),
        mesh=mesh,
        compiler_params=_sc_params(),
        scratch_types=(
            [pltpu.VMEM((_M,), jnp.int32) for _ in range(_K)]
            + [pltpu.VMEM((_M,), jnp.int32) for _ in range(_K)]
            + [pltpu.VMEM((_G,), jnp.float32),
               pltpu.VMEM((_G, hdim), jnp.float32),
               pltpu.VMEM_SHARED((n, hdim), jnp.float32)]
        ),
    )
    def kern(col_hbm, row_hbm, ew_hbm, h_hbm, out_hbm, *scr):
        col_v = scr[:_K]
        row_v = scr[_K:2 * _K]
        ew_v, rows_v, agg_sh = scr[2 * _K:]
        c = lax.axis_index("c")
        s = lax.axis_index("s")
        wid = c * _NS + s
        hoff = pl.multiple_of(s * stripe, 8)
        pltpu.sync_copy(h_hbm.at[pl.ds(hoff, stripe)],
                        agg_sh.at[pl.ds(s * stripe, stripe)])
        plsc.subcore_barrier()

        my_blocks = (nblocks - 1 - wid) // _NW + 1

        @pl.loop(0, my_blocks)
        def _chunk(ci):
            b = wid + ci * _NW
            e0 = pl.multiple_of(b * _G, 8)
            pltpu.sync_copy(ew_hbm.at[pl.ds(e0, _G)], ew_v)
            for j in range(_K):
                ej = pl.multiple_of(b * _G + j * _M, 8)
                pltpu.sync_copy(col_hbm.at[pl.ds(ej, _M)], col_v[j])
                pltpu.sync_copy(row_hbm.at[pl.ds(ej, _M)], row_v[j])
            for j in range(_K):
                pltpu.sync_copy(h_hbm.at[col_v[j]],
                                rows_v.at[pl.ds(j * _M, _M)])

            @pl.loop(0, _G // _L)
            def _grp(g):
                g0 = g * _L
                ew16 = ew_v[pl.ds(g0, _L)]
                for k in range(_L):
                    w = _lane_bcast(ew16, k)
                    r = g0 + k
                    rows_v[r, pl.ds(0, _L)] = rows_v[r, pl.ds(0, _L)] * w
                    rows_v[r, pl.ds(_L, _L)] = rows_v[r, pl.ds(_L, _L)] * w

            for j in range(_K):
                pltpu.sync_copy(rows_v.at[pl.ds(j * _M, _M)],
                                agg_sh.at[row_v[j]], add=True)

        plsc.subcore_barrier()
        pltpu.sync_copy(agg_sh.at[pl.ds(s * stripe, stripe)],
                        out_hbm.at[c].at[pl.ds(hoff, stripe)])

    return kern(col, row, ew, h)


def _degrees(row, ew, ones, npad):
    e = ew.shape[0]
    nblocks = e // _G
    dstripe = npad // _NS
    mesh = plsc.VectorSubcoreMesh(core_axis_name="c", subcore_axis_name="s")

    @functools.partial(
        pl.kernel,
        out_type=jax.ShapeDtypeStruct((_NC, npad), jnp.float32),
        mesh=mesh,
        compiler_params=_sc_params(),
        scratch_types=(
            [pltpu.VMEM((_M,), jnp.int32) for _ in range(_K)]
            + [pltpu.VMEM((_M,), jnp.float32) for _ in range(_K)]
            + [pltpu.VMEM_SHARED((npad,), jnp.float32)]
        ),
    )
    def kern(row_hbm, ew_hbm, ones_hbm, out_hbm, *scr):
        row_v = scr[:_K]
        ew_v = scr[_K:2 * _K]
        deg_sh = scr[2 * _K]
        c = lax.axis_index("c")
        s = lax.axis_index("s")
        wid = c * _NS + s
        doff = pl.multiple_of(s * dstripe, 8)
        pltpu.sync_copy(ones_hbm.at[pl.ds(doff, dstripe)],
                        deg_sh.at[pl.ds(s * dstripe, dstripe)])
        plsc.subcore_barrier()

        my_blocks = (nblocks - 1 - wid) // _NW + 1

        @pl.loop(0, my_blocks)
        def _chunk(ci):
            b = wid + ci * _NW
            for j in range(_K):
                ej = pl.multiple_of(b * _G + j * _M, 8)
                pltpu.sync_copy(row_hbm.at[pl.ds(ej, _M)], row_v[j])
                pltpu.sync_copy(ew_hbm.at[pl.ds(ej, _M)], ew_v[j])
            for j in range(_K):
                pltpu.sync_copy(ew_v[j], deg_sh.at[row_v[j]], add=True)
        del _chunk

        plsc.subcore_barrier()
        pltpu.sync_copy(deg_sh.at[pl.ds(s * dstripe, dstripe)],
                        out_hbm.at[c].at[pl.ds(doff, dstripe)])

    return kern(row, ew, ones)


def _input_proj(x, w_in, b_in):
    t, n, f = x.shape
    hdim = w_in.shape[0]

    def body(x_ref, w_ref, b_ref, o_ref):
        xb = x_ref[0]
        o_ref[0] = (lax.dot_general(xb, w_ref[...], (((1,), (1,)), ((), ())),
                                    preferred_element_type=jnp.float32)
                    + b_ref[...])

    return pl.pallas_call(
        body,
        grid=(t,),
        in_specs=[
            pl.BlockSpec((1, n, f), lambda i: (i, 0, 0)),
            pl.BlockSpec((hdim, f), lambda i: (0, 0)),
            pl.BlockSpec((1, hdim), lambda i: (0, 0)),
        ],
        out_specs=pl.BlockSpec((1, n, hdim), lambda i: (i, 0, 0)),
        out_shape=jax.ShapeDtypeStruct((t, n, hdim), jnp.float32),
    )(x, w_in, b_in.reshape(1, hdim))


def _update(xproj_t, h, aggp, degp, w_h):
    n, hdim = h.shape

    def body(xp_ref, h_ref, ag_ref, dg_ref, w_ref, o_ref):
        hcur = h_ref[...]
        raw = ag_ref[0] + ag_ref[1] - hcur
        deg = dg_ref[0] + dg_ref[1] - 1.0
        tot = raw * (1.0 / deg)[:, None]
        pre = xp_ref[...] + lax.dot_general(
            tot, w_ref[...], (((1,), (1,)), ((), ())),
            preferred_element_type=jnp.float32)
        o_ref[...] = (1.0 - _LEAK) * hcur + _LEAK * jnp.tanh(pre)

    return pl.pallas_call(
        body,
        out_shape=jax.ShapeDtypeStruct((n, hdim), jnp.float32),
    )(xproj_t, h, aggp, degp, w_h)


def kernel(x, edge_index, edge_weight, W_in, b_in, W_h):
    t, n, _ = x.shape
    hdim = W_h.shape[0]
    e = edge_weight.shape[0]

    col = edge_index[0].astype(jnp.int32)
    row = edge_index[1].astype(jnp.int32)
    ew = edge_weight.astype(jnp.float32)

    npad = ((n + 8 * _NS - 1) // (8 * _NS)) * (8 * _NS)
    ones = jnp.ones((npad,), jnp.float32)

    xproj = _input_proj(x, W_in, b_in)
    xproj = jnp.pad(xproj, ((0, 0), (0, npad - n), (0, 0)))
    degp = _degrees(row, ew, ones, npad)

    zeros_h = jnp.zeros((npad, hdim), jnp.float32)
    zeros_agg = jnp.zeros((_NC, npad, hdim), jnp.float32)
    zeros_deg = jnp.zeros((_NC, npad), jnp.float32)

    h = _update(xproj[0], zeros_h, zeros_agg, zeros_deg, W_h)
    outs = [h]
    for step in range(1, t):
        aggp = _propagate(col, row, ew, h)
        h = _update(xproj[step], h, aggp, degp, W_h)
        outs.append(h)
    return jnp.stack(outs)[:, :n, :]


# speedup vs baseline: 30.4829x; 3.1742x over previous
import dataclasses
import functools

import jax
import jax.numpy as jnp
from jax import lax
from jax.experimental import pallas as pl
from jax.experimental.pallas import tpu as pltpu
from jax.experimental.pallas import tpu_sc as plsc

_LEAK = 0.9

_NC = 2
_NS = 16
_NW = _NC * _NS
_L = 16

def _sc_params():
    cp = pltpu.CompilerParams()
    if "use_tc_tiling_on_sc" in pltpu.CompilerParams.__dataclass_fields__:
        cp = dataclasses.replace(cp, use_tc_tiling_on_sc=False)
    return cp


_M = 80
_K = 8
_G = _M * _K

_PM = 128
_PK = 8
_PG = _PM * _PK


def _lane_bcast(vec, lane):
    idx = jnp.full((_L,), lane, dtype=jnp.int32)
    dnums = lax.GatherDimensionNumbers(
        offset_dims=(), collapsed_slice_dims=(0,), start_index_map=(0,))
    return lax.gather(vec, idx[:, None], dnums, slice_sizes=(1,),
                      mode=lax.GatherScatterMode.PROMISE_IN_BOUNDS)


def _propagate(col, row2d, ew, h):
    n, hdim = h.shape
    epad = ew.shape[0]
    nb = epad // (_PG * _NW)
    stripe = n // _NS
    mesh = plsc.VectorSubcoreMesh(core_axis_name="c", subcore_axis_name="s")

    @functools.partial(
        pl.kernel,
        out_type=jax.ShapeDtypeStruct((_NC, n, hdim), jnp.float32),
        mesh=mesh,
        compiler_params=_sc_params(),
        scratch_types=(
            [pltpu.VMEM((_PG,), jnp.int32) for _ in range(2)]
            + [pltpu.VMEM((_PK, _PM), jnp.int32) for _ in range(4)]
            + [pltpu.VMEM((_PG,), jnp.float32) for _ in range(2)]
            + [pltpu.VMEM((_PG, hdim), jnp.float32) for _ in range(2)]
            + [pltpu.SemaphoreType.DMA for _ in range(6)]
            + [pltpu.VMEM_SHARED((n, hdim), jnp.float32)]
        ),
    )
    def kern(col_hbm, row_hbm, ew_hbm, h_hbm, out_hbm, *scr):
        cols = scr[0:2]
        rowi = scr[2:6]
        ews = scr[6:8]
        rowsd = scr[8:10]
        semi = scr[10:12]
        semg = scr[12:14]
        sems = scr[14:16]
        agg_sh = scr[16]
        c = lax.axis_index("c")
        s = lax.axis_index("s")
        wid = c * _NS + s
        hoff = pl.multiple_of(s * stripe, 8)
        pltpu.sync_copy(h_hbm.at[pl.ds(hoff, stripe)],
                        agg_sh.at[pl.ds(s * stripe, stripe)])
        plsc.subcore_barrier()

        def issue_idx(ci):
            b = wid + ci * _NW
            p = ci % 2
            e0 = pl.multiple_of(b * _PG, 8)
            r0 = pl.multiple_of(b * _PK, 8)
            return [
                pltpu.async_copy(ew_hbm.at[pl.ds(e0, _PG)], ews[p], semi[p]),
                pltpu.async_copy(col_hbm.at[pl.ds(e0, _PG)], cols[p], semi[p]),
                pltpu.async_copy(row_hbm.at[pl.ds(r0, _PK)], rowi[ci % 4],
                                 semi[p]),
            ]

        def issue_gather(ci):
            p = ci % 2
            return [
                pltpu.async_copy(h_hbm.at[cols[p].at[pl.ds(j * _PM, _PM)]],
                                 rowsd[p].at[pl.ds(j * _PM, _PM)], semg[p])
                for j in range(_PK)
            ]

        def issue_scatter(ci):
            p = ci % 2
            return [
                pltpu.async_copy(rowsd[p].at[pl.ds(j * _PM, _PM)],
                                 agg_sh.at[rowi[ci % 4].at[j]], sems[p],
                                 add=True)
                for j in range(_PK)
            ]

        def scale(p):
            @pl.loop(0, _PG // _L)
            def _grp(g):
                g0 = g * _L
                ew16 = ews[p][pl.ds(g0, _L)]
                for k in range(_L):
                    w = _lane_bcast(ew16, k)
                    r = g0 + k
                    rowsd[p][r, pl.ds(0, _L)] = rowsd[p][r, pl.ds(0, _L)] * w
                    rowsd[p][r, pl.ds(_L, _L)] = (
                        rowsd[p][r, pl.ds(_L, _L)] * w)

        def wait(handles):
            for hh in handles:
                hh.wait()

        idx_h = [None] * (nb + 2)
        gat_h = [None] * (nb + 1)
        sca_h = [None] * nb
        idx_h[0] = issue_idx(0)
        if nb > 1:
            idx_h[1] = issue_idx(1)
        wait(idx_h[0])
        gat_h[0] = issue_gather(0)
        for ci in range(nb):
            p = ci % 2
            wait(gat_h[ci])
            if ci + 1 < nb:
                if ci >= 1:
                    wait(sca_h[ci - 1])
                wait(idx_h[ci + 1])
                gat_h[ci + 1] = issue_gather(ci + 1)
            scale(p)
            sca_h[ci] = issue_scatter(ci)
            if ci + 2 < nb:
                idx_h[ci + 2] = issue_idx(ci + 2)
        if nb >= 2:
            wait(sca_h[nb - 2])
        wait(sca_h[nb - 1])

        plsc.subcore_barrier()
        pltpu.sync_copy(agg_sh.at[pl.ds(s * stripe, stripe)],
                        out_hbm.at[c].at[pl.ds(hoff, stripe)])

    return kern(col, row2d, ew, h)


def _degrees(row, ew, ones, npad):
    e = ew.shape[0]
    nblocks = e // _G
    dstripe = npad // _NS
    mesh = plsc.VectorSubcoreMesh(core_axis_name="c", subcore_axis_name="s")

    @functools.partial(
        pl.kernel,
        out_type=jax.ShapeDtypeStruct((_NC, npad), jnp.float32),
        mesh=mesh,
        compiler_params=_sc_params(),
        scratch_types=(
            [pltpu.VMEM((_M,), jnp.int32) for _ in range(_K)]
            + [pltpu.VMEM((_M,), jnp.float32) for _ in range(_K)]
            + [pltpu.VMEM_SHARED((npad,), jnp.float32)]
        ),
    )
    def kern(row_hbm, ew_hbm, ones_hbm, out_hbm, *scr):
        row_v = scr[:_K]
        ew_v = scr[_K:2 * _K]
        deg_sh = scr[2 * _K]
        c = lax.axis_index("c")
        s = lax.axis_index("s")
        wid = c * _NS + s
        doff = pl.multiple_of(s * dstripe, 8)
        pltpu.sync_copy(ones_hbm.at[pl.ds(doff, dstripe)],
                        deg_sh.at[pl.ds(s * dstripe, dstripe)])
        plsc.subcore_barrier()

        my_blocks = (nblocks - 1 - wid) // _NW + 1

        @pl.loop(0, my_blocks)
        def _chunk(ci):
            b = wid + ci * _NW
            for j in range(_K):
                ej = pl.multiple_of(b * _G + j * _M, 8)
                pltpu.sync_copy(row_hbm.at[pl.ds(ej, _M)], row_v[j])
                pltpu.sync_copy(ew_hbm.at[pl.ds(ej, _M)], ew_v[j])
            for j in range(_K):
                pltpu.sync_copy(ew_v[j], deg_sh.at[row_v[j]], add=True)
        del _chunk

        plsc.subcore_barrier()
        pltpu.sync_copy(deg_sh.at[pl.ds(s * dstripe, dstripe)],
                        out_hbm.at[c].at[pl.ds(doff, dstripe)])

    return kern(row, ew, ones)


def _input_proj(x, w_in, b_in):
    t, n, f = x.shape
    hdim = w_in.shape[0]

    def body(x_ref, w_ref, b_ref, o_ref):
        xb = x_ref[0]
        o_ref[0] = (lax.dot_general(xb, w_ref[...], (((1,), (1,)), ((), ())),
                                    preferred_element_type=jnp.float32)
                    + b_ref[...])

    return pl.pallas_call(
        body,
        grid=(t,),
        in_specs=[
            pl.BlockSpec((1, n, f), lambda i: (i, 0, 0)),
            pl.BlockSpec((hdim, f), lambda i: (0, 0)),
            pl.BlockSpec((1, hdim), lambda i: (0, 0)),
        ],
        out_specs=pl.BlockSpec((1, n, hdim), lambda i: (i, 0, 0)),
        out_shape=jax.ShapeDtypeStruct((t, n, hdim), jnp.float32),
    )(x, w_in, b_in.reshape(1, hdim))


def _update(xproj_t, h, aggp, degp, w_h):
    n, hdim = h.shape

    def body(xp_ref, h_ref, ag_ref, dg_ref, w_ref, o_ref):
        hcur = h_ref[...]
        raw = ag_ref[0] + ag_ref[1] - hcur
        deg = dg_ref[0] + dg_ref[1] - 1.0
        tot = raw * (1.0 / deg)[:, None]
        pre = xp_ref[...] + lax.dot_general(
            tot, w_ref[...], (((1,), (1,)), ((), ())),
            preferred_element_type=jnp.float32)
        o_ref[...] = (1.0 - _LEAK) * hcur + _LEAK * jnp.tanh(pre)

    return pl.pallas_call(
        body,
        out_shape=jax.ShapeDtypeStruct((n, hdim), jnp.float32),
    )(xproj_t, h, aggp, degp, w_h)


def kernel(x, edge_index, edge_weight, W_in, b_in, W_h):
    t, n, _ = x.shape
    hdim = W_h.shape[0]
    e = edge_weight.shape[0]

    col = edge_index[0].astype(jnp.int32)
    row = edge_index[1].astype(jnp.int32)
    ew = edge_weight.astype(jnp.float32)

    block = _PG * _NW
    epad = ((e + block - 1) // block) * block
    if epad != e:
        fill = jnp.arange(epad - e, dtype=jnp.int32) % n
        col = jnp.concatenate([col, fill])
        row = jnp.concatenate([row, fill])
        ew = jnp.concatenate([ew, jnp.zeros((epad - e,), jnp.float32)])
    row2d = row.reshape(epad // _PM, _PM)

    npad = ((n + 8 * _NS - 1) // (8 * _NS)) * (8 * _NS)
    ones = jnp.ones((npad,), jnp.float32)

    xproj = _input_proj(x, W_in, b_in)
    xproj = jnp.pad(xproj, ((0, 0), (0, npad - n), (0, 0)))
    degp = _degrees(row, ew, ones, npad)

    zeros_h = jnp.zeros((npad, hdim), jnp.float32)
    zeros_agg = jnp.zeros((_NC, npad, hdim), jnp.float32)
    zeros_deg = jnp.zeros((_NC, npad), jnp.float32)

    h = _update(xproj[0], zeros_h, zeros_agg, zeros_deg, W_h)
    outs = [h]
    for step in range(1, t):
        aggp = _propagate(col, row2d, ew, h)
        h = _update(xproj[step], h, aggp, degp, W_h)
        outs.append(h)
    return jnp.stack(outs)[:, :n, :]


# speedup vs baseline: 34.6300x; 1.1360x over previous
import dataclasses
import functools

import jax
import jax.numpy as jnp
from jax import lax
from jax.experimental import pallas as pl
from jax.experimental.pallas import tpu as pltpu
from jax.experimental.pallas import tpu_sc as plsc

_LEAK = 0.9

_NC = 2
_NS = 16
_NW = _NC * _NS
_L = 16

def _sc_params():
    cp = pltpu.CompilerParams()
    if "use_tc_tiling_on_sc" in pltpu.CompilerParams.__dataclass_fields__:
        cp = dataclasses.replace(cp, use_tc_tiling_on_sc=False)
    return cp


_M = 80
_K = 8
_G = _M * _K

_PM = 128
_PK = 8
_PG = _PM * _PK


def _lane_bcast(vec, lane):
    idx = jnp.full((_L,), lane, dtype=jnp.int32)
    dnums = lax.GatherDimensionNumbers(
        offset_dims=(), collapsed_slice_dims=(0,), start_index_map=(0,))
    return lax.gather(vec, idx[:, None], dnums, slice_sizes=(1,),
                      mode=lax.GatherScatterMode.PROMISE_IN_BOUNDS)


def _propagate(col, row2d, ew, h):
    n, hdim = h.shape
    epad = ew.shape[0]
    nb = epad // (_PG * _NW)
    stripe = n // _NS
    mesh = plsc.VectorSubcoreMesh(core_axis_name="c", subcore_axis_name="s")

    @functools.partial(
        pl.kernel,
        out_type=jax.ShapeDtypeStruct((_NC, n, hdim), jnp.float32),
        mesh=mesh,
        compiler_params=_sc_params(),
        scratch_types=(
            [pltpu.VMEM((_PG,), jnp.int32) for _ in range(3)]
            + [pltpu.VMEM((_PK, _PM), jnp.int32) for _ in range(4)]
            + [pltpu.VMEM((_PG,), jnp.float32) for _ in range(3)]
            + [pltpu.VMEM((_PG, hdim), jnp.float32) for _ in range(3)]
            + [pltpu.SemaphoreType.DMA for _ in range(9)]
            + [pltpu.VMEM_SHARED((n, hdim), jnp.float32)]
        ),
    )
    def kern(col_hbm, row_hbm, ew_hbm, h_hbm, out_hbm, *scr):
        cols = scr[0:3]
        rowi = scr[3:7]
        ews = scr[7:10]
        rowsd = scr[10:13]
        semi = scr[13:16]
        semg = scr[16:19]
        sems = scr[19:22]
        agg_sh = scr[22]
        c = lax.axis_index("c")
        s = lax.axis_index("s")
        wid = c * _NS + s
        hoff = pl.multiple_of(s * stripe, 8)
        pltpu.sync_copy(h_hbm.at[pl.ds(hoff, stripe)],
                        agg_sh.at[pl.ds(s * stripe, stripe)])
        plsc.subcore_barrier()

        def issue_idx(ci):
            b = wid + ci * _NW
            p = ci % 3
            e0 = pl.multiple_of(b * _PG, 8)
            r0 = pl.multiple_of(b * _PK, 8)
            return [
                pltpu.async_copy(ew_hbm.at[pl.ds(e0, _PG)], ews[p], semi[p]),
                pltpu.async_copy(col_hbm.at[pl.ds(e0, _PG)], cols[p], semi[p]),
                pltpu.async_copy(row_hbm.at[pl.ds(r0, _PK)], rowi[ci % 4],
                                 semi[p]),
            ]

        def issue_gather(ci):
            p = ci % 3
            return [
                pltpu.async_copy(h_hbm.at[cols[p].at[pl.ds(j * _PM, _PM)]],
                                 rowsd[p].at[pl.ds(j * _PM, _PM)], semg[p])
                for j in range(_PK)
            ]

        def issue_scatter(ci):
            p = ci % 3
            return [
                pltpu.async_copy(rowsd[p].at[pl.ds(j * _PM, _PM)],
                                 agg_sh.at[rowi[ci % 4].at[j]], sems[p],
                                 add=True)
                for j in range(_PK)
            ]

        def scale(p):
            @pl.loop(0, _PG // _L)
            def _grp(g):
                g0 = g * _L
                ew16 = ews[p][pl.ds(g0, _L)]
                for k in range(_L):
                    w = _lane_bcast(ew16, k)
                    r = g0 + k
                    rowsd[p][r, pl.ds(0, _L)] = rowsd[p][r, pl.ds(0, _L)] * w
                    rowsd[p][r, pl.ds(_L, _L)] = (
                        rowsd[p][r, pl.ds(_L, _L)] * w)

        def wait(handles):
            for hh in handles:
                hh.wait()

        idx_h = [None] * (nb + 2)
        gat_h = [None] * (nb + 1)
        sca_h = [None] * nb
        idx_h[0] = issue_idx(0)
        if nb > 1:
            idx_h[1] = issue_idx(1)
        wait(idx_h[0])
        gat_h[0] = issue_gather(0)
        for ci in range(nb):
            wait(gat_h[ci])
            if ci >= 2:
                wait(sca_h[ci - 2])
            if ci + 1 < nb:
                wait(idx_h[ci + 1])
                gat_h[ci + 1] = issue_gather(ci + 1)
            scale(ci % 3)
            sca_h[ci] = issue_scatter(ci)
            if ci + 2 < nb:
                idx_h[ci + 2] = issue_idx(ci + 2)
        if nb >= 2:
            wait(sca_h[nb - 2])
        wait(sca_h[nb - 1])

        plsc.subcore_barrier()
        pltpu.sync_copy(agg_sh.at[pl.ds(s * stripe, stripe)],
                        out_hbm.at[c].at[pl.ds(hoff, stripe)])

    return kern(col, row2d, ew, h)


def _degrees(row2d, ew2d, ones, npad):
    nrows = row2d.shape[0]
    nb = nrows // (_PK * _NW)
    dstripe = npad // _NS
    mesh = plsc.VectorSubcoreMesh(core_axis_name="c", subcore_axis_name="s")

    @functools.partial(
        pl.kernel,
        out_type=jax.ShapeDtypeStruct((_NC, npad), jnp.float32),
        mesh=mesh,
        compiler_params=_sc_params(),
        scratch_types=[
            pltpu.VMEM((nb, _PK, _PM), jnp.int32),
            pltpu.VMEM((nb, _PK, _PM), jnp.float32),
            pltpu.SemaphoreType.DMA,
            pltpu.SemaphoreType.DMA,
            pltpu.VMEM_SHARED((npad,), jnp.float32),
        ],
    )
    def kern(row_hbm, ew_hbm, ones_hbm, out_hbm, row_v, ew_v, semi, sems,
             deg_sh):
        c = lax.axis_index("c")
        s = lax.axis_index("s")
        wid = c * _NS + s
        doff = pl.multiple_of(s * dstripe, 8)
        loads = []
        for b in range(nb):
            r0 = pl.multiple_of((wid + b * _NW) * _PK, 8)
            loads.append(pltpu.async_copy(row_hbm.at[pl.ds(r0, _PK)],
                                          row_v.at[b], semi))
            loads.append(pltpu.async_copy(ew_hbm.at[pl.ds(r0, _PK)],
                                          ew_v.at[b], semi))
        pltpu.sync_copy(ones_hbm.at[pl.ds(doff, dstripe)],
                        deg_sh.at[pl.ds(s * dstripe, dstripe)])
        plsc.subcore_barrier()
        for hh in loads:
            hh.wait()
        scats = []
        for b in range(nb):
            for j in range(_PK):
                scats.append(pltpu.async_copy(
                    ew_v.at[b].at[j], deg_sh.at[row_v.at[b].at[j]], sems,
                    add=True))
        for hh in scats:
            hh.wait()

        plsc.subcore_barrier()
        pltpu.sync_copy(deg_sh.at[pl.ds(s * dstripe, dstripe)],
                        out_hbm.at[c].at[pl.ds(doff, dstripe)])

    return kern(row2d, ew2d, ones)


def _input_proj(x, w_in, b_in):
    t, n, f = x.shape
    hdim = w_in.shape[0]

    def body(x_ref, w_ref, b_ref, o_ref):
        xb = x_ref[0]
        o_ref[0] = (lax.dot_general(xb, w_ref[...], (((1,), (1,)), ((), ())),
                                    preferred_element_type=jnp.float32)
                    + b_ref[...])

    return pl.pallas_call(
        body,
        grid=(t,),
        in_specs=[
            pl.BlockSpec((1, n, f), lambda i: (i, 0, 0)),
            pl.BlockSpec((hdim, f), lambda i: (0, 0)),
            pl.BlockSpec((1, hdim), lambda i: (0, 0)),
        ],
        out_specs=pl.BlockSpec((1, n, hdim), lambda i: (i, 0, 0)),
        out_shape=jax.ShapeDtypeStruct((t, n, hdim), jnp.float32),
    )(x, w_in, b_in.reshape(1, hdim))


def _update(xproj_t, h, aggp, degp, w_h):
    n, hdim = h.shape

    def body(xp_ref, h_ref, ag_ref, dg_ref, w_ref, o_ref):
        hcur = h_ref[...]
        raw = ag_ref[0] + ag_ref[1] - hcur
        deg = dg_ref[0] + dg_ref[1] - 1.0
        tot = raw * (1.0 / deg)[:, None]
        pre = xp_ref[...] + lax.dot_general(
            tot, w_ref[...], (((1,), (1,)), ((), ())),
            preferred_element_type=jnp.float32)
        o_ref[...] = (1.0 - _LEAK) * hcur + _LEAK * jnp.tanh(pre)

    return pl.pallas_call(
        body,
        out_shape=jax.ShapeDtypeStruct((n, hdim), jnp.float32),
    )(xproj_t, h, aggp, degp, w_h)


def kernel(x, edge_index, edge_weight, W_in, b_in, W_h):
    t, n, _ = x.shape
    hdim = W_h.shape[0]
    e = edge_weight.shape[0]

    col = edge_index[0].astype(jnp.int32)
    row = edge_index[1].astype(jnp.int32)
    ew = edge_weight.astype(jnp.float32)

    block = _PG * _NW
    epad = ((e + block - 1) // block) * block
    if epad != e:
        fill = jnp.arange(epad - e, dtype=jnp.int32) % n
        col = jnp.concatenate([col, fill])
        row = jnp.concatenate([row, fill])
        ew = jnp.concatenate([ew, jnp.zeros((epad - e,), jnp.float32)])
    row2d = row.reshape(epad // _PM, _PM)
    ew2d = ew.reshape(epad // _PM, _PM)

    npad = ((n + 8 * _NS - 1) // (8 * _NS)) * (8 * _NS)
    ones = jnp.ones((npad,), jnp.float32)

    xproj = _input_proj(x, W_in, b_in)
    xproj = jnp.pad(xproj, ((0, 0), (0, npad - n), (0, 0)))
    degp = _degrees(row2d, ew2d, ones, npad)

    zeros_h = jnp.zeros((npad, hdim), jnp.float32)
    zeros_agg = jnp.zeros((_NC, npad, hdim), jnp.float32)
    zeros_deg = jnp.zeros((_NC, npad), jnp.float32)

    h = _update(xproj[0], zeros_h, zeros_agg, zeros_deg, W_h)
    outs = [h]
    for step in range(1, t):
        aggp = _propagate(col, row2d, ew, h)
        h = _update(xproj[step], h, aggp, degp, W_h)
        outs.append(h)
    return jnp.stack(outs)[:, :n, :]
